# Initial kernel scaffold; baseline (speedup 1.0000x reference)
import functools

import jax
import jax.numpy as jnp
from jax import lax
from jax.experimental import pallas as pl
from jax.experimental.pallas import tpu as pltpu
from jax.experimental.pallas import tpu_sc as plsc

N_NODES = 10000
N_EDGES = 320000
D = 128

NC = 2
NS = 16
NW = NC * NS
EDGES_PER_TILE = N_EDGES // NW
CHUNK = 80
N_CHUNKS = EDGES_PER_TILE // CHUNK
ROWS_PER_TILE = N_NODES // NS
ZROWS = 25


def _sc_scatter_partials(edge_attr, receivers):
    mesh = plsc.VectorSubcoreMesh(core_axis_name="c", subcore_axis_name="s")

    @functools.partial(
        pl.kernel,
        out_type=jax.ShapeDtypeStruct((NC, N_NODES, D), jnp.float32),
        mesh=mesh,
        scratch_types=[
            pltpu.VMEM((CHUNK, D), jnp.float32),
            pltpu.VMEM((CHUNK,), jnp.int32),
            pltpu.VMEM((ZROWS, D), jnp.float32),
            pltpu.VMEM_SHARED((N_NODES, D), jnp.float32),
        ],
    )
    def body(ea_hbm, recv_hbm, out_hbm, ebuf, ibuf, zbuf, acc):
        cid = lax.axis_index("c")
        sid = lax.axis_index("s")
        wid = cid * NS + sid

        for j in range(ZROWS):
            for k in range(D // 16):
                zbuf[j, pl.ds(k * 16, 16)] = jnp.zeros((16,), jnp.float32)

        row0 = sid * ROWS_PER_TILE

        def zloop(r, carry):
            pltpu.sync_copy(zbuf, acc.at[pl.ds(row0 + r * ZROWS, ZROWS)])
            return carry

        lax.fori_loop(0, ROWS_PER_TILE // ZROWS, zloop, 0)
        plsc.subcore_barrier()

        base0 = wid * EDGES_PER_TILE

        def chunk_body(i, carry):
            base = base0 + i * CHUNK
            pltpu.sync_copy(recv_hbm.at[pl.ds(base, CHUNK)], ibuf)
            pltpu.sync_copy(ea_hbm.at[pl.ds(base, CHUNK)], ebuf)
            pltpu.sync_copy(ebuf, acc.at[ibuf], add=True)
            return carry

        lax.fori_loop(0, N_CHUNKS, chunk_body, 0)
        plsc.subcore_barrier()

        pltpu.sync_copy(acc.at[pl.ds(row0, ROWS_PER_TILE)],
                        out_hbm.at[cid, pl.ds(row0, ROWS_PER_TILE)])

    return body(edge_attr, receivers)


BLK = 1000


def _mlp_body(x_ref, p_ref, w0_ref, b0_ref, w1_ref, b1_ref, w2_ref, b2_ref,
              o_ref):
    agg = p_ref[0] + p_ref[1]
    inp = jnp.concatenate([x_ref[...], agg], axis=-1)
    h = jnp.dot(inp, w0_ref[...], preferred_element_type=jnp.float32)
    h = jnp.maximum(h + b0_ref[...], 0.0)
    h = jnp.dot(h, w1_ref[...], preferred_element_type=jnp.float32)
    h = jnp.maximum(h + b1_ref[...], 0.0)
    h = jnp.dot(h, w2_ref[...], preferred_element_type=jnp.float32)
    h = h + b2_ref[...]
    mean = jnp.mean(h, axis=-1, keepdims=True)
    var = jnp.mean((h - mean) ** 2, axis=-1, keepdims=True)
    o_ref[...] = (h - mean) * lax.rsqrt(var + 1e-5)


def _tc_mlp(x, partials, W0, b0, W1, b1, W2, b2):
    grid = N_NODES // BLK
    return pl.pallas_call(
        _mlp_body,
        grid=(grid,),
        in_specs=[
            pl.BlockSpec((BLK, D), lambda i: (i, 0)),
            pl.BlockSpec((NC, BLK, D), lambda i: (0, i, 0)),
            pl.BlockSpec((2 * D, D), lambda i: (0, 0)),
            pl.BlockSpec((D,), lambda i: (0,)),
            pl.BlockSpec((D, D), lambda i: (0, 0)),
            pl.BlockSpec((D,), lambda i: (0,)),
            pl.BlockSpec((D, D), lambda i: (0, 0)),
            pl.BlockSpec((D,), lambda i: (0,)),
        ],
        out_specs=pl.BlockSpec((BLK, D), lambda i: (i, 0)),
        out_shape=jax.ShapeDtypeStruct((N_NODES, D), jnp.float32),
    )(x, partials, W0, b0, W1, b1, W2, b2)


def kernel(x, edge_attr, receivers, senders, W0, b0, W1, b1, W2, b2):
    partials = _sc_scatter_partials(edge_attr, receivers.astype(jnp.int32))
    updated_nodes = _tc_mlp(x, partials, W0, b0, W1, b1, W2, b2)
    return (updated_nodes, edge_attr, receivers, senders)


# speedup vs baseline: 3.0207x; 3.0207x over previous
import functools

import jax
import jax.numpy as jnp
from jax import lax
from jax.experimental import pallas as pl
from jax.experimental.pallas import tpu as pltpu
from jax.experimental.pallas import tpu_sc as plsc

N_NODES = 10000
N_EDGES = 320000
D = 128

NC = 2
NS = 16
NW = NC * NS
EDGES_PER_TILE = N_EDGES // NW
CHUNK = 80
N_CHUNKS = EDGES_PER_TILE // CHUNK
N_PAD = 10240
ROWS_PER_TILE = N_PAD // NS
ZROWS = 40


def _sc_scatter_partials(edge_attr, receivers):
    mesh = plsc.VectorSubcoreMesh(core_axis_name="c", subcore_axis_name="s")

    @functools.partial(
        pl.kernel,
        out_type=jax.ShapeDtypeStruct((NC, N_PAD, D), jnp.float32),
        mesh=mesh,
        scratch_types=[
            pltpu.VMEM((CHUNK, D), jnp.float32),
            pltpu.VMEM((CHUNK,), jnp.int32),
            pltpu.VMEM((ZROWS, D), jnp.float32),
            pltpu.VMEM_SHARED((N_PAD, D), jnp.float32),
        ],
    )
    def body(ea_hbm, recv_hbm, out_hbm, ebuf, ibuf, zbuf, acc):
        cid = lax.axis_index("c")
        sid = lax.axis_index("s")
        wid = cid * NS + sid

        for j in range(ZROWS):
            for k in range(D // 16):
                zbuf[j, pl.ds(k * 16, 16)] = jnp.zeros((16,), jnp.float32)

        row0 = sid * ROWS_PER_TILE

        def zloop(r, carry):
            pltpu.sync_copy(zbuf, acc.at[pl.ds(row0 + r * ZROWS, ZROWS)])
            return carry

        lax.fori_loop(0, ROWS_PER_TILE // ZROWS, zloop, 0)
        plsc.subcore_barrier()

        base0 = wid * EDGES_PER_TILE

        def chunk_body(i, carry):
            base = base0 + i * CHUNK
            pltpu.sync_copy(recv_hbm.at[pl.ds(base, CHUNK)], ibuf)
            pltpu.sync_copy(ea_hbm.at[pl.ds(base, CHUNK)], ebuf)
            pltpu.sync_copy(ebuf, acc.at[ibuf], add=True)
            return carry

        lax.fori_loop(0, N_CHUNKS, chunk_body, 0)
        plsc.subcore_barrier()

        pltpu.sync_copy(acc.at[pl.ds(row0, ROWS_PER_TILE)],
                        out_hbm.at[cid, pl.ds(row0, ROWS_PER_TILE)])

    return body(edge_attr, receivers)


BLK = 1000


def _mlp_body(x_ref, p_ref, w0_ref, b0_ref, w1_ref, b1_ref, w2_ref, b2_ref,
              o_ref):
    agg = p_ref[0] + p_ref[1]
    inp = jnp.concatenate([x_ref[...], agg], axis=-1)
    h = jnp.dot(inp, w0_ref[...], preferred_element_type=jnp.float32)
    h = jnp.maximum(h + b0_ref[...], 0.0)
    h = jnp.dot(h, w1_ref[...], preferred_element_type=jnp.float32)
    h = jnp.maximum(h + b1_ref[...], 0.0)
    h = jnp.dot(h, w2_ref[...], preferred_element_type=jnp.float32)
    h = h + b2_ref[...]
    mean = jnp.mean(h, axis=-1, keepdims=True)
    var = jnp.mean((h - mean) ** 2, axis=-1, keepdims=True)
    o_ref[...] = (h - mean) * lax.rsqrt(var + 1e-5)


def _tc_mlp(x, partials, W0, b0, W1, b1, W2, b2):
    grid = N_NODES // BLK
    return pl.pallas_call(
        _mlp_body,
        grid=(grid,),
        in_specs=[
            pl.BlockSpec((BLK, D), lambda i: (i, 0)),
            pl.BlockSpec((NC, BLK, D), lambda i: (0, i, 0)),
            pl.BlockSpec((2 * D, D), lambda i: (0, 0)),
            pl.BlockSpec((D,), lambda i: (0,)),
            pl.BlockSpec((D, D), lambda i: (0, 0)),
            pl.BlockSpec((D,), lambda i: (0,)),
            pl.BlockSpec((D, D), lambda i: (0, 0)),
            pl.BlockSpec((D,), lambda i: (0,)),
        ],
        out_specs=pl.BlockSpec((BLK, D), lambda i: (i, 0)),
        out_shape=jax.ShapeDtypeStruct((N_NODES, D), jnp.float32),
    )(x, partials, W0, b0, W1, b1, W2, b2)


def kernel(x, edge_attr, receivers, senders, W0, b0, W1, b1, W2, b2):
    partials = _sc_scatter_partials(edge_attr, receivers.astype(jnp.int32))
    updated_nodes = _tc_mlp(x, partials, W0, b0, W1, b1, W2, b2)
    return (updated_nodes, edge_attr, receivers, senders)


# speedup vs baseline: 4.1203x; 1.3640x over previous
import functools

import jax
import jax.numpy as jnp
from jax import lax
from jax.experimental import pallas as pl
from jax.experimental.pallas import tpu as pltpu
from jax.experimental.pallas import tpu_sc as plsc

N_NODES = 10000
N_EDGES = 320000
D = 128

NC = 2
NS = 16
NW = NC * NS
EDGES_PER_TILE = N_EDGES // NW
CHUNK = 80
N_CHUNKS = EDGES_PER_TILE // CHUNK
N_PAD = 10240
ROWS_PER_TILE = N_PAD // NS
ZROWS = 40


def _sc_scatter_partials(edge_attr, receivers):
    mesh = plsc.VectorSubcoreMesh(core_axis_name="c", subcore_axis_name="s")

    @functools.partial(
        pl.kernel,
        out_type=jax.ShapeDtypeStruct((NC, N_PAD, D), jnp.float32),
        mesh=mesh,
        scratch_types=[
            pltpu.VMEM((CHUNK, D), jnp.float32),
            pltpu.VMEM((CHUNK, D), jnp.float32),
            pltpu.VMEM((CHUNK,), jnp.int32),
            pltpu.VMEM((CHUNK,), jnp.int32),
            pltpu.SemaphoreType.DMA,
            pltpu.SemaphoreType.DMA,
            pltpu.VMEM((ZROWS, D), jnp.float32),
            pltpu.VMEM_SHARED((N_PAD, D), jnp.float32),
        ],
    )
    def body(ea_hbm, recv_hbm, out_hbm, ebuf0, ebuf1, ibuf0, ibuf1,
             sem0, sem1, zbuf, acc):
        cid = lax.axis_index("c")
        sid = lax.axis_index("s")
        wid = cid * NS + sid

        for j in range(ZROWS):
            for k in range(D // 16):
                zbuf[j, pl.ds(k * 16, 16)] = jnp.zeros((16,), jnp.float32)

        row0 = sid * ROWS_PER_TILE

        def zloop(r, carry):
            pltpu.sync_copy(zbuf, acc.at[pl.ds(row0 + r * ZROWS, ZROWS)])
            return carry

        lax.fori_loop(0, ROWS_PER_TILE // ZROWS, zloop, 0)
        plsc.subcore_barrier()

        base0 = wid * EDGES_PER_TILE
        bufs = ((ebuf0, ibuf0, sem0), (ebuf1, ibuf1, sem1))

        def start_loads(i, slot):
            eb, ib, sem = bufs[slot]
            base = base0 + i * CHUNK
            pltpu.async_copy(recv_hbm.at[pl.ds(base, CHUNK)], ib, sem)
            pltpu.async_copy(ea_hbm.at[pl.ds(base, CHUNK)], eb, sem)

        def wait_loads(i, slot):
            eb, ib, sem = bufs[slot]
            base = base0 + i * CHUNK
            pltpu.make_async_copy(recv_hbm.at[pl.ds(base, CHUNK)], ib,
                                  sem).wait()
            pltpu.make_async_copy(ea_hbm.at[pl.ds(base, CHUNK)], eb,
                                  sem).wait()

        def scatter(slot):
            eb, ib, _ = bufs[slot]
            pltpu.sync_copy(eb, acc.at[ib], add=True)

        start_loads(0, 0)

        def chunk_pair(t, carry):
            i0 = 2 * t
            wait_loads(i0, 0)
            start_loads(i0 + 1, 1)
            scatter(0)
            wait_loads(i0 + 1, 1)

            @pl.when(i0 + 2 < N_CHUNKS)
            def _():
                start_loads(i0 + 2, 0)

            scatter(1)
            return carry

        lax.fori_loop(0, (N_CHUNKS - 1) // 2, chunk_pair, 0)
        wait_loads(N_CHUNKS - 1, 0)
        scatter(0)
        plsc.subcore_barrier()

        pltpu.sync_copy(acc.at[pl.ds(row0, ROWS_PER_TILE)],
                        out_hbm.at[cid, pl.ds(row0, ROWS_PER_TILE)])

    return body(edge_attr, receivers)


BLK = 1000


def _mlp_body(x_ref, p_ref, w0_ref, b0_ref, w1_ref, b1_ref, w2_ref, b2_ref,
              o_ref):
    agg = p_ref[0] + p_ref[1]
    inp = jnp.concatenate([x_ref[...], agg], axis=-1)
    h = jnp.dot(inp, w0_ref[...], preferred_element_type=jnp.float32)
    h = jnp.maximum(h + b0_ref[...], 0.0)
    h = jnp.dot(h, w1_ref[...], preferred_element_type=jnp.float32)
    h = jnp.maximum(h + b1_ref[...], 0.0)
    h = jnp.dot(h, w2_ref[...], preferred_element_type=jnp.float32)
    h = h + b2_ref[...]
    mean = jnp.mean(h, axis=-1, keepdims=True)
    var = jnp.mean((h - mean) ** 2, axis=-1, keepdims=True)
    o_ref[...] = (h - mean) * lax.rsqrt(var + 1e-5)


def _tc_mlp(x, partials, W0, b0, W1, b1, W2, b2):
    grid = N_NODES // BLK
    return pl.pallas_call(
        _mlp_body,
        grid=(grid,),
        in_specs=[
            pl.BlockSpec((BLK, D), lambda i: (i, 0)),
            pl.BlockSpec((NC, BLK, D), lambda i: (0, i, 0)),
            pl.BlockSpec((2 * D, D), lambda i: (0, 0)),
            pl.BlockSpec((D,), lambda i: (0,)),
            pl.BlockSpec((D, D), lambda i: (0, 0)),
            pl.BlockSpec((D,), lambda i: (0,)),
            pl.BlockSpec((D, D), lambda i: (0, 0)),
            pl.BlockSpec((D,), lambda i: (0,)),
        ],
        out_specs=pl.BlockSpec((BLK, D), lambda i: (i, 0)),
        out_shape=jax.ShapeDtypeStruct((N_NODES, D), jnp.float32),
    )(x, partials, W0, b0, W1, b1, W2, b2)


def kernel(x, edge_attr, receivers, senders, W0, b0, W1, b1, W2, b2):
    partials = _sc_scatter_partials(edge_attr, receivers.astype(jnp.int32))
    updated_nodes = _tc_mlp(x, partials, W0, b0, W1, b1, W2, b2)
    return (updated_nodes, edge_attr, receivers, senders)


# speedup vs baseline: 4.8700x; 1.1820x over previous
import functools

import jax
import jax.numpy as jnp
from jax import lax
from jax.experimental import pallas as pl
from jax.experimental.pallas import tpu as pltpu
from jax.experimental.pallas import tpu_sc as plsc

N_NODES = 10000
N_EDGES = 320000
D = 128

NC = 2
NS = 16
NW = NC * NS
EDGES_PER_TILE = N_EDGES // NW
CHUNK = 80
N_CHUNKS = EDGES_PER_TILE // CHUNK
N_PAD = 10240
ROWS_PER_TILE = N_PAD // NS
ZROWS = 40


def _sc_scatter_partials(edge_attr, receivers):
    mesh = plsc.VectorSubcoreMesh(core_axis_name="c", subcore_axis_name="s")

    @functools.partial(
        pl.kernel,
        out_type=jax.ShapeDtypeStruct((NC, N_PAD, D), jnp.float32),
        mesh=mesh,
        scratch_types=[
            pltpu.VMEM((CHUNK, D), jnp.float32),
            pltpu.VMEM((CHUNK, D), jnp.float32),
            pltpu.VMEM((CHUNK,), jnp.int32),
            pltpu.VMEM((CHUNK,), jnp.int32),
            pltpu.SemaphoreType.DMA,
            pltpu.SemaphoreType.DMA,
            pltpu.VMEM((ZROWS, D), jnp.float32),
            pltpu.VMEM_SHARED((N_PAD, D), jnp.float32),
        ],
    )
    def body(ea_hbm, recv_hbm, out_hbm, ebuf0, ebuf1, ibuf0, ibuf1,
             sem0, sem1, zbuf, acc):
        cid = lax.axis_index("c")
        sid = lax.axis_index("s")
        wid = cid * NS + sid

        for j in range(ZROWS):
            for k in range(D // 16):
                zbuf[j, pl.ds(k * 16, 16)] = jnp.zeros((16,), jnp.float32)

        row0 = sid * ROWS_PER_TILE

        def zloop(r, carry):
            pltpu.sync_copy(zbuf, acc.at[pl.ds(row0 + r * ZROWS, ZROWS)])
            return carry

        lax.fori_loop(0, ROWS_PER_TILE // ZROWS, zloop, 0)
        plsc.subcore_barrier()

        base0 = wid * EDGES_PER_TILE
        bufs = ((ebuf0, ibuf0, sem0), (ebuf1, ibuf1, sem1))

        def start_loads(i, slot):
            eb, ib, sem = bufs[slot]
            base = base0 + i * CHUNK
            pltpu.async_copy(recv_hbm.at[pl.ds(base, CHUNK)], ib, sem)
            pltpu.async_copy(ea_hbm.at[pl.ds(base, CHUNK)], eb, sem)

        def wait_loads(i, slot):
            eb, ib, sem = bufs[slot]
            base = base0 + i * CHUNK
            pltpu.make_async_copy(recv_hbm.at[pl.ds(base, CHUNK)], ib,
                                  sem).wait()
            pltpu.make_async_copy(ea_hbm.at[pl.ds(base, CHUNK)], eb,
                                  sem).wait()

        def scatter(slot):
            eb, ib, _ = bufs[slot]
            pltpu.sync_copy(eb, acc.at[ib], add=True)

        start_loads(0, 0)

        def chunk_pair(t, carry):
            i0 = 2 * t
            wait_loads(i0, 0)
            start_loads(i0 + 1, 1)
            scatter(0)
            wait_loads(i0 + 1, 1)

            @pl.when(i0 + 2 < N_CHUNKS)
            def _():
                start_loads(i0 + 2, 0)

            scatter(1)
            return carry

        lax.fori_loop(0, (N_CHUNKS - 1) // 2, chunk_pair, 0)
        wait_loads(N_CHUNKS - 1, 0)
        scatter(0)
        plsc.subcore_barrier()

        pltpu.sync_copy(acc.at[pl.ds(row0, ROWS_PER_TILE)],
                        out_hbm.at[cid, pl.ds(row0, ROWS_PER_TILE)])

    return body(edge_attr, receivers)


BLK = 1000


def _mlp_body(x_ref, p_ref, w0_ref, b0_ref, w1_ref, b1_ref, w2_ref, b2_ref,
              o_ref):
    agg = p_ref[0] + p_ref[1]
    inp = jnp.concatenate([x_ref[...], agg], axis=-1)
    h = jnp.dot(inp, w0_ref[...], preferred_element_type=jnp.float32)
    h = jnp.maximum(h + b0_ref[...], 0.0)
    h = jnp.dot(h, w1_ref[...], preferred_element_type=jnp.float32)
    h = jnp.maximum(h + b1_ref[...], 0.0)
    h = jnp.dot(h, w2_ref[...], preferred_element_type=jnp.float32)
    h = h + b2_ref[...]
    mean = jnp.mean(h, axis=-1, keepdims=True)
    var = jnp.mean((h - mean) ** 2, axis=-1, keepdims=True)
    o_ref[...] = (h - mean) * lax.rsqrt(var + 1e-5)


def _tc_mlp(x, partials, W0, b0, W1, b1, W2, b2):
    grid = N_NODES // BLK
    return pl.pallas_call(
        _mlp_body,
        grid=(grid,),
        in_specs=[
            pl.BlockSpec((BLK, D), lambda i: (i, 0)),
            pl.BlockSpec((NC, BLK, D), lambda i: (0, i, 0)),
            pl.BlockSpec((2 * D, D), lambda i: (0, 0)),
            pl.BlockSpec((D,), lambda i: (0,)),
            pl.BlockSpec((D, D), lambda i: (0, 0)),
            pl.BlockSpec((D,), lambda i: (0,)),
            pl.BlockSpec((D, D), lambda i: (0, 0)),
            pl.BlockSpec((D,), lambda i: (0,)),
        ],
        out_specs=pl.BlockSpec((BLK, D), lambda i: (i, 0)),
        out_shape=jax.ShapeDtypeStruct((N_NODES, D), jnp.float32),
    )(x, partials, W0, b0, W1, b1, W2, b2)


CBLK = 4000


def _copy_body(e_ref, o_ref):
    o_ref[...] = e_ref[...]


def _tc_copy_edges(edge_attr):
    return pl.pallas_call(
        _copy_body,
        grid=(N_EDGES // CBLK,),
        in_specs=[pl.BlockSpec((CBLK, D), lambda i: (i, 0))],
        out_specs=pl.BlockSpec((CBLK, D), lambda i: (i, 0)),
        out_shape=jax.ShapeDtypeStruct((N_EDGES, D), jnp.float32),
    )(edge_attr)


def kernel(x, edge_attr, receivers, senders, W0, b0, W1, b1, W2, b2):
    partials = _sc_scatter_partials(edge_attr, receivers.astype(jnp.int32))
    edge_attr_out = _tc_copy_edges(edge_attr)
    updated_nodes = _tc_mlp(x, partials, W0, b0, W1, b1, W2, b2)
    return (updated_nodes, edge_attr_out, receivers, senders)


# speedup vs baseline: 5.4525x; 1.1196x over previous
import functools

import jax
import jax.numpy as jnp
from jax import lax
from jax.experimental import pallas as pl
from jax.experimental.pallas import tpu as pltpu
from jax.experimental.pallas import tpu_sc as plsc

N_NODES = 10000
N_EDGES = 320000
D = 128

NC = 2
NS = 16
NW = NC * NS
EDGES_PER_TILE = N_EDGES // NW
CHUNK = 80
N_CHUNKS = EDGES_PER_TILE // CHUNK
N_PAD = 10240
ROWS_PER_TILE = N_PAD // NS
ZROWS = 40


def _sc_scatter_partials(edge_attr, receivers):
    mesh = plsc.VectorSubcoreMesh(core_axis_name="c", subcore_axis_name="s")

    @functools.partial(
        pl.kernel,
        out_type=(jax.ShapeDtypeStruct((NC, N_PAD, D), jnp.float32),
                  jax.ShapeDtypeStruct((N_EDGES, D), jnp.float32)),
        mesh=mesh,
        scratch_types=[
            pltpu.VMEM((CHUNK, D), jnp.float32),
            pltpu.VMEM((CHUNK, D), jnp.float32),
            pltpu.VMEM((CHUNK,), jnp.int32),
            pltpu.VMEM((CHUNK,), jnp.int32),
            pltpu.SemaphoreType.DMA,
            pltpu.SemaphoreType.DMA,
            pltpu.SemaphoreType.DMA,
            pltpu.SemaphoreType.DMA,
            pltpu.VMEM((ZROWS, D), jnp.float32),
            pltpu.VMEM_SHARED((N_PAD, D), jnp.float32),
        ],
    )
    def body(ea_hbm, recv_hbm, out_hbm, eout_hbm, ebuf0, ebuf1, ibuf0, ibuf1,
             sem0, sem1, ssem0, ssem1, zbuf, acc):
        cid = lax.axis_index("c")
        sid = lax.axis_index("s")
        wid = cid * NS + sid

        for j in range(ZROWS):
            for k in range(D // 16):
                zbuf[j, pl.ds(k * 16, 16)] = jnp.zeros((16,), jnp.float32)

        row0 = sid * ROWS_PER_TILE

        def zloop(r, carry):
            pltpu.sync_copy(zbuf, acc.at[pl.ds(row0 + r * ZROWS, ZROWS)])
            return carry

        lax.fori_loop(0, ROWS_PER_TILE // ZROWS, zloop, 0)
        plsc.subcore_barrier()

        base0 = wid * EDGES_PER_TILE
        bufs = ((ebuf0, ibuf0, sem0, ssem0), (ebuf1, ibuf1, sem1, ssem1))

        def start_loads(i, slot):
            eb, ib, sem, _ = bufs[slot]
            base = base0 + i * CHUNK
            pltpu.async_copy(recv_hbm.at[pl.ds(base, CHUNK)], ib, sem)
            pltpu.async_copy(ea_hbm.at[pl.ds(base, CHUNK)], eb, sem)

        def wait_loads(i, slot):
            eb, ib, sem, _ = bufs[slot]
            base = base0 + i * CHUNK
            pltpu.make_async_copy(recv_hbm.at[pl.ds(base, CHUNK)], ib,
                                  sem).wait()
            pltpu.make_async_copy(ea_hbm.at[pl.ds(base, CHUNK)], eb,
                                  sem).wait()

        def scatter(slot):
            eb, ib, _, _ = bufs[slot]
            pltpu.sync_copy(eb, acc.at[ib], add=True)

        def start_store(i, slot):
            eb, _, _, ssem = bufs[slot]
            base = base0 + i * CHUNK
            pltpu.async_copy(eb, eout_hbm.at[pl.ds(base, CHUNK)], ssem)

        def wait_store(i, slot):
            eb, _, _, ssem = bufs[slot]
            base = base0 + i * CHUNK
            pltpu.make_async_copy(eb, eout_hbm.at[pl.ds(base, CHUNK)],
                                  ssem).wait()

        start_loads(0, 0)

        def chunk_pair(t, carry):
            i0 = 2 * t
            wait_loads(i0, 0)

            @pl.when(t > 0)
            def _():
                wait_store(i0 - 1, 1)

            start_loads(i0 + 1, 1)
            scatter(0)
            start_store(i0, 0)
            wait_loads(i0 + 1, 1)
            wait_store(i0, 0)

            @pl.when(i0 + 2 < N_CHUNKS)
            def _():
                start_loads(i0 + 2, 0)

            scatter(1)
            start_store(i0 + 1, 1)
            return carry

        lax.fori_loop(0, (N_CHUNKS - 1) // 2, chunk_pair, 0)
        wait_loads(N_CHUNKS - 1, 0)
        wait_store(N_CHUNKS - 2, 1)
        scatter(0)
        start_store(N_CHUNKS - 1, 0)
        wait_store(N_CHUNKS - 1, 0)
        plsc.subcore_barrier()

        pltpu.sync_copy(acc.at[pl.ds(row0, ROWS_PER_TILE)],
                        out_hbm.at[cid, pl.ds(row0, ROWS_PER_TILE)])

    return body(edge_attr, receivers)


BLK = 1000


def _mlp_body(x_ref, p_ref, w0_ref, b0_ref, w1_ref, b1_ref, w2_ref, b2_ref,
              o_ref):
    agg = p_ref[0] + p_ref[1]
    inp = jnp.concatenate([x_ref[...], agg], axis=-1)
    h = jnp.dot(inp, w0_ref[...], preferred_element_type=jnp.float32)
    h = jnp.maximum(h + b0_ref[...], 0.0)
    h = jnp.dot(h, w1_ref[...], preferred_element_type=jnp.float32)
    h = jnp.maximum(h + b1_ref[...], 0.0)
    h = jnp.dot(h, w2_ref[...], preferred_element_type=jnp.float32)
    h = h + b2_ref[...]
    mean = jnp.mean(h, axis=-1, keepdims=True)
    var = jnp.mean((h - mean) ** 2, axis=-1, keepdims=True)
    o_ref[...] = (h - mean) * lax.rsqrt(var + 1e-5)


def _tc_mlp(x, partials, W0, b0, W1, b1, W2, b2):
    grid = N_NODES // BLK
    return pl.pallas_call(
        _mlp_body,
        grid=(grid,),
        in_specs=[
            pl.BlockSpec((BLK, D), lambda i: (i, 0)),
            pl.BlockSpec((NC, BLK, D), lambda i: (0, i, 0)),
            pl.BlockSpec((2 * D, D), lambda i: (0, 0)),
            pl.BlockSpec((D,), lambda i: (0,)),
            pl.BlockSpec((D, D), lambda i: (0, 0)),
            pl.BlockSpec((D,), lambda i: (0,)),
            pl.BlockSpec((D, D), lambda i: (0, 0)),
            pl.BlockSpec((D,), lambda i: (0,)),
        ],
        out_specs=pl.BlockSpec((BLK, D), lambda i: (i, 0)),
        out_shape=jax.ShapeDtypeStruct((N_NODES, D), jnp.float32),
    )(x, partials, W0, b0, W1, b1, W2, b2)


def kernel(x, edge_attr, receivers, senders, W0, b0, W1, b1, W2, b2):
    partials, edge_attr_out = _sc_scatter_partials(
        edge_attr, receivers.astype(jnp.int32))
    updated_nodes = _tc_mlp(x, partials, W0, b0, W1, b1, W2, b2)
    return (updated_nodes, edge_attr_out, receivers, senders)


# speedup vs baseline: 6.0407x; 1.1079x over previous
import functools

import jax
import jax.numpy as jnp
from jax import lax
from jax.experimental import pallas as pl
from jax.experimental.pallas import tpu as pltpu
from jax.experimental.pallas import tpu_sc as plsc

N_NODES = 10000
N_EDGES = 320000
D = 128

NC = 2
NS = 16
NW = NC * NS
EDGES_PER_TILE = N_EDGES // NW
CHUNK = 128
N_CHUNKS = EDGES_PER_TILE // CHUNK
TAIL = EDGES_PER_TILE - N_CHUNKS * CHUNK
NSLOT = 2
N_PAD = 10240
ROWS_PER_TILE = N_PAD // NS
ZROWS = 32


def _sc_scatter_partials(edge_attr, receivers):
    mesh = plsc.VectorSubcoreMesh(core_axis_name="c", subcore_axis_name="s")

    @functools.partial(
        pl.kernel,
        out_type=(jax.ShapeDtypeStruct((NC, N_PAD, D), jnp.float32),
                  jax.ShapeDtypeStruct((N_EDGES, D), jnp.float32)),
        mesh=mesh,
        scratch_types=(
            [pltpu.VMEM((CHUNK, D), jnp.float32)] * NSLOT
            + [pltpu.VMEM((CHUNK,), jnp.int32)] * NSLOT
            + [pltpu.SemaphoreType.DMA] * NSLOT
            + [pltpu.SemaphoreType.DMA] * NSLOT
            + [pltpu.SemaphoreType.DMA] * NSLOT
            + [
                pltpu.VMEM((TAIL, D), jnp.float32),
                pltpu.VMEM((TAIL,), jnp.int32),
                pltpu.VMEM((ZROWS, D), jnp.float32),
                pltpu.VMEM_SHARED((N_PAD, D), jnp.float32),
            ]
        ),
    )
    def body(ea_hbm, recv_hbm, out_hbm, eout_hbm,
             eb0, eb1, ib0, ib1, lsem0, lsem1,
             csem0, csem1, ssem0, ssem1,
             tbuf, tibuf, zbuf, acc):
        cid = lax.axis_index("c")
        sid = lax.axis_index("s")
        wid = cid * NS + sid

        for j in range(ZROWS):
            for k in range(D // 16):
                zbuf[j, pl.ds(k * 16, 16)] = jnp.zeros((16,), jnp.float32)

        row0 = sid * ROWS_PER_TILE

        def zloop(r, carry):
            pltpu.sync_copy(zbuf, acc.at[pl.ds(row0 + r * ZROWS, ZROWS)])
            return carry

        lax.fori_loop(0, ROWS_PER_TILE // ZROWS, zloop, 0)
        plsc.subcore_barrier()

        base0 = wid * EDGES_PER_TILE
        ebufs = (eb0, eb1)
        ibufs = (ib0, ib1)
        lsems = (lsem0, lsem1)
        csems = (csem0, csem1)
        ssems = (ssem0, ssem1)

        def start_loads(i, s):
            base = base0 + i * CHUNK
            pltpu.async_copy(recv_hbm.at[pl.ds(base, CHUNK)], ibufs[s],
                             lsems[s])
            pltpu.async_copy(ea_hbm.at[pl.ds(base, CHUNK)], ebufs[s],
                             lsems[s])

        def wait_loads(i, s):
            base = base0 + i * CHUNK
            pltpu.make_async_copy(recv_hbm.at[pl.ds(base, CHUNK)], ibufs[s],
                                  lsems[s]).wait()
            pltpu.make_async_copy(ea_hbm.at[pl.ds(base, CHUNK)], ebufs[s],
                                  lsems[s]).wait()

        def start_scatter(s):
            pltpu.async_copy(ebufs[s], acc.at[ibufs[s]], csems[s], add=True)

        def wait_scatter(s):
            pltpu.make_async_copy(ebufs[s], acc.at[ibufs[s]],
                                  csems[s]).wait()

        def start_store(i, s):
            base = base0 + i * CHUNK
            pltpu.async_copy(ebufs[s], eout_hbm.at[pl.ds(base, CHUNK)],
                             ssems[s])

        def wait_store(i, s):
            base = base0 + i * CHUNK
            pltpu.make_async_copy(ebufs[s], eout_hbm.at[pl.ds(base, CHUNK)],
                                  ssems[s]).wait()

        for s in range(NSLOT):
            start_loads(s, s)

        def ring_body(t, carry):
            c = NSLOT * t
            for s in range(NSLOT):
                wait_loads(c + s, s)
                start_scatter(s)
                start_store(c + s, s)
            for s in range(NSLOT):
                wait_scatter(s)
                wait_store(c + s, s)

                @pl.when(c + s + NSLOT < N_CHUNKS)
                def _(s=s, c=c):
                    start_loads(c + s + NSLOT, s)

            return carry

        lax.fori_loop(0, N_CHUNKS // NSLOT, ring_body, 0)
        tbase = base0 + N_CHUNKS * CHUNK
        pltpu.sync_copy(recv_hbm.at[pl.ds(tbase, TAIL)], tibuf)
        pltpu.sync_copy(ea_hbm.at[pl.ds(tbase, TAIL)], tbuf)
        pltpu.sync_copy(tbuf, acc.at[tibuf], add=True)
        pltpu.sync_copy(tbuf, eout_hbm.at[pl.ds(tbase, TAIL)])
        plsc.subcore_barrier()

        pltpu.sync_copy(acc.at[pl.ds(row0, ROWS_PER_TILE)],
                        out_hbm.at[cid, pl.ds(row0, ROWS_PER_TILE)])

    return body(edge_attr, receivers)


BLK = 1000


def _mlp_body(x_ref, p_ref, w0_ref, b0_ref, w1_ref, b1_ref, w2_ref, b2_ref,
              o_ref):
    agg = p_ref[0] + p_ref[1]
    inp = jnp.concatenate([x_ref[...], agg], axis=-1)
    h = jnp.dot(inp, w0_ref[...], preferred_element_type=jnp.float32)
    h = jnp.maximum(h + b0_ref[...], 0.0)
    h = jnp.dot(h, w1_ref[...], preferred_element_type=jnp.float32)
    h = jnp.maximum(h + b1_ref[...], 0.0)
    h = jnp.dot(h, w2_ref[...], preferred_element_type=jnp.float32)
    h = h + b2_ref[...]
    mean = jnp.mean(h, axis=-1, keepdims=True)
    var = jnp.mean((h - mean) ** 2, axis=-1, keepdims=True)
    o_ref[...] = (h - mean) * lax.rsqrt(var + 1e-5)


def _tc_mlp(x, partials, W0, b0, W1, b1, W2, b2):
    grid = N_NODES // BLK
    return pl.pallas_call(
        _mlp_body,
        grid=(grid,),
        in_specs=[
            pl.BlockSpec((BLK, D), lambda i: (i, 0)),
            pl.BlockSpec((NC, BLK, D), lambda i: (0, i, 0)),
            pl.BlockSpec((2 * D, D), lambda i: (0, 0)),
            pl.BlockSpec((D,), lambda i: (0,)),
            pl.BlockSpec((D, D), lambda i: (0, 0)),
            pl.BlockSpec((D,), lambda i: (0,)),
            pl.BlockSpec((D, D), lambda i: (0, 0)),
            pl.BlockSpec((D,), lambda i: (0,)),
        ],
        out_specs=pl.BlockSpec((BLK, D), lambda i: (i, 0)),
        out_shape=jax.ShapeDtypeStruct((N_NODES, D), jnp.float32),
    )(x, partials, W0, b0, W1, b1, W2, b2)


def kernel(x, edge_attr, receivers, senders, W0, b0, W1, b1, W2, b2):
    partials, edge_attr_out = _sc_scatter_partials(
        edge_attr, receivers.astype(jnp.int32))
    updated_nodes = _tc_mlp(x, partials, W0, b0, W1, b1, W2, b2)
    return (updated_nodes, edge_attr_out, receivers, senders)


# speedup vs baseline: 6.1231x; 1.0136x over previous
import functools

import jax
import jax.numpy as jnp
from jax import lax
from jax.experimental import pallas as pl
from jax.experimental.pallas import tpu as pltpu
from jax.experimental.pallas import tpu_sc as plsc

N_NODES = 10000
N_EDGES = 320000
D = 128

NC = 2
NS = 16
NW = NC * NS
EDGES_PER_TILE = N_EDGES // NW
CHUNK = 128
N_CHUNKS = EDGES_PER_TILE // CHUNK
TAIL = EDGES_PER_TILE - N_CHUNKS * CHUNK
NSLOT = 2
N_PAD = 10240
ROWS_PER_TILE = N_PAD // NS
ZROWS = 32


def _sc_scatter_partials(edge_attr, receivers):
    mesh = plsc.VectorSubcoreMesh(core_axis_name="c", subcore_axis_name="s")

    @functools.partial(
        pl.kernel,
        out_type=(jax.ShapeDtypeStruct((NC, N_PAD, D), jnp.float32),
                  jax.ShapeDtypeStruct((N_EDGES, D), jnp.float32)),
        mesh=mesh,
        scratch_types=(
            [pltpu.VMEM((CHUNK, D), jnp.float32)] * NSLOT
            + [pltpu.VMEM((CHUNK,), jnp.int32)] * NSLOT
            + [pltpu.SemaphoreType.DMA] * NSLOT
            + [pltpu.SemaphoreType.DMA] * NSLOT
            + [pltpu.SemaphoreType.DMA] * NSLOT
            + [
                pltpu.VMEM((TAIL, D), jnp.float32),
                pltpu.VMEM((TAIL,), jnp.int32),
                pltpu.VMEM((ZROWS, D), jnp.float32),
                pltpu.VMEM_SHARED((N_PAD, D), jnp.float32),
            ]
        ),
    )
    def body(ea_hbm, recv_hbm, out_hbm, eout_hbm,
             eb0, eb1, ib0, ib1, lsem0, lsem1,
             csem0, csem1, ssem0, ssem1,
             tbuf, tibuf, zbuf, acc):
        cid = lax.axis_index("c")
        sid = lax.axis_index("s")
        wid = cid * NS + sid
        row0 = sid * ROWS_PER_TILE
        base0 = wid * EDGES_PER_TILE
        ebufs = (eb0, eb1)
        ibufs = (ib0, ib1)
        lsems = (lsem0, lsem1)
        csems = (csem0, csem1)
        ssems = (ssem0, ssem1)

        def start_loads(i, s):
            base = base0 + i * CHUNK
            pltpu.async_copy(recv_hbm.at[pl.ds(base, CHUNK)], ibufs[s],
                             lsems[s])
            pltpu.async_copy(ea_hbm.at[pl.ds(base, CHUNK)], ebufs[s],
                             lsems[s])

        def wait_loads(i, s):
            base = base0 + i * CHUNK
            pltpu.make_async_copy(recv_hbm.at[pl.ds(base, CHUNK)], ibufs[s],
                                  lsems[s]).wait()
            pltpu.make_async_copy(ea_hbm.at[pl.ds(base, CHUNK)], ebufs[s],
                                  lsems[s]).wait()

        def start_scatter(s):
            pltpu.async_copy(ebufs[s], acc.at[ibufs[s]], csems[s], add=True)

        def wait_scatter(s):
            pltpu.make_async_copy(ebufs[s], acc.at[ibufs[s]],
                                  csems[s]).wait()

        def start_store(i, s):
            base = base0 + i * CHUNK
            pltpu.async_copy(ebufs[s], eout_hbm.at[pl.ds(base, CHUNK)],
                             ssems[s])

        def wait_store(i, s):
            base = base0 + i * CHUNK
            pltpu.make_async_copy(ebufs[s], eout_hbm.at[pl.ds(base, CHUNK)],
                                  ssems[s]).wait()

        for s in range(NSLOT):
            start_loads(s, s)

        for j in range(ZROWS):
            for k in range(D // 16):
                zbuf[j, pl.ds(k * 16, 16)] = jnp.zeros((16,), jnp.float32)

        nz = ROWS_PER_TILE // ZROWS

        def zfire(r, carry):
            pltpu.async_copy(zbuf, acc.at[pl.ds(row0 + r * ZROWS, ZROWS)],
                             csem0)
            return carry

        lax.fori_loop(0, nz, zfire, 0)

        def zdrain(r, carry):
            pltpu.make_async_copy(zbuf, acc.at[pl.ds(row0 + r * ZROWS, ZROWS)],
                                  csem0).wait()
            return carry

        lax.fori_loop(0, nz, zdrain, 0)
        plsc.subcore_barrier()

        def ring_body(t, carry):
            c = NSLOT * t
            for s in range(NSLOT):
                wait_loads(c + s, s)
                start_scatter(s)
                start_store(c + s, s)
            for s in range(NSLOT):
                wait_scatter(s)
                wait_store(c + s, s)

                @pl.when(c + s + NSLOT < N_CHUNKS)
                def _(s=s, c=c):
                    start_loads(c + s + NSLOT, s)

            return carry

        lax.fori_loop(0, N_CHUNKS // NSLOT, ring_body, 0)
        tbase = base0 + N_CHUNKS * CHUNK
        pltpu.sync_copy(recv_hbm.at[pl.ds(tbase, TAIL)], tibuf)
        pltpu.sync_copy(ea_hbm.at[pl.ds(tbase, TAIL)], tbuf)
        pltpu.sync_copy(tbuf, acc.at[tibuf], add=True)
        pltpu.sync_copy(tbuf, eout_hbm.at[pl.ds(tbase, TAIL)])
        plsc.subcore_barrier()

        pltpu.sync_copy(acc.at[pl.ds(row0, ROWS_PER_TILE)],
                        out_hbm.at[cid, pl.ds(row0, ROWS_PER_TILE)])

    return body(edge_attr, receivers)


BLK = 1000


def _mlp_body(x_ref, p_ref, w0_ref, b0_ref, w1_ref, b1_ref, w2_ref, b2_ref,
              o_ref):
    agg = p_ref[0] + p_ref[1]
    inp = jnp.concatenate([x_ref[...], agg], axis=-1)
    h = jnp.dot(inp, w0_ref[...], preferred_element_type=jnp.float32)
    h = jnp.maximum(h + b0_ref[...], 0.0)
    h = jnp.dot(h, w1_ref[...], preferred_element_type=jnp.float32)
    h = jnp.maximum(h + b1_ref[...], 0.0)
    h = jnp.dot(h, w2_ref[...], preferred_element_type=jnp.float32)
    h = h + b2_ref[...]
    mean = jnp.mean(h, axis=-1, keepdims=True)
    var = jnp.mean((h - mean) ** 2, axis=-1, keepdims=True)
    o_ref[...] = (h - mean) * lax.rsqrt(var + 1e-5)


def _tc_mlp(x, partials, W0, b0, W1, b1, W2, b2):
    grid = N_NODES // BLK
    return pl.pallas_call(
        _mlp_body,
        grid=(grid,),
        in_specs=[
            pl.BlockSpec((BLK, D), lambda i: (i, 0)),
            pl.BlockSpec((NC, BLK, D), lambda i: (0, i, 0)),
            pl.BlockSpec((2 * D, D), lambda i: (0, 0)),
            pl.BlockSpec((D,), lambda i: (0,)),
            pl.BlockSpec((D, D), lambda i: (0, 0)),
            pl.BlockSpec((D,), lambda i: (0,)),
            pl.BlockSpec((D, D), lambda i: (0, 0)),
            pl.BlockSpec((D,), lambda i: (0,)),
        ],
        out_specs=pl.BlockSpec((BLK, D), lambda i: (i, 0)),
        out_shape=jax.ShapeDtypeStruct((N_NODES, D), jnp.float32),
    )(x, partials, W0, b0, W1, b1, W2, b2)


def kernel(x, edge_attr, receivers, senders, W0, b0, W1, b1, W2, b2):
    partials, edge_attr_out = _sc_scatter_partials(
        edge_attr, receivers.astype(jnp.int32))
    updated_nodes = _tc_mlp(x, partials, W0, b0, W1, b1, W2, b2)
    return (updated_nodes, edge_attr_out, receivers, senders)


# speedup vs baseline: 6.5964x; 1.0773x over previous
import functools

import jax
import jax.numpy as jnp
from jax import lax
from jax.experimental import pallas as pl
from jax.experimental.pallas import tpu as pltpu
from jax.experimental.pallas import tpu_sc as plsc

N_NODES = 10000
N_EDGES = 320000
D = 128

NC = 2
NS = 16
NW = NC * NS
EDGES_PER_TILE = N_EDGES // NW
CHUNK = 104
N_CHUNKS = EDGES_PER_TILE // CHUNK
TAIL = EDGES_PER_TILE - N_CHUNKS * CHUNK
NSLOT = 3
N_PAD = 10240
ROWS_PER_TILE = N_PAD // NS
ZROWS = 32


def _sc_scatter_partials(edge_attr, receivers):
    mesh = plsc.VectorSubcoreMesh(core_axis_name="c", subcore_axis_name="s")

    @functools.partial(
        pl.kernel,
        out_type=(jax.ShapeDtypeStruct((NC, N_PAD, D), jnp.float32),
                  jax.ShapeDtypeStruct((N_EDGES, D), jnp.float32)),
        mesh=mesh,
        scratch_types=(
            [pltpu.VMEM((CHUNK, D), jnp.float32)] * NSLOT
            + [pltpu.VMEM((CHUNK,), jnp.int32)] * NSLOT
            + [pltpu.SemaphoreType.DMA] * NSLOT
            + [pltpu.SemaphoreType.DMA] * NSLOT
            + [pltpu.SemaphoreType.DMA] * NSLOT
            + [
                pltpu.VMEM((TAIL, D), jnp.float32),
                pltpu.VMEM((TAIL,), jnp.int32),
                pltpu.VMEM((ZROWS, D), jnp.float32),
                pltpu.VMEM_SHARED((N_PAD, D), jnp.float32),
            ]
        ),
    )
    def body(ea_hbm, recv_hbm, out_hbm, eout_hbm,
             eb0, eb1, eb2, ib0, ib1, ib2, lsem0, lsem1, lsem2,
             csem0, csem1, csem2, ssem0, ssem1, ssem2,
             tbuf, tibuf, zbuf, acc):
        cid = lax.axis_index("c")
        sid = lax.axis_index("s")
        wid = cid * NS + sid
        row0 = sid * ROWS_PER_TILE
        base0 = wid * EDGES_PER_TILE
        ebufs = (eb0, eb1, eb2)
        ibufs = (ib0, ib1, ib2)
        lsems = (lsem0, lsem1, lsem2)
        csems = (csem0, csem1, csem2)
        ssems = (ssem0, ssem1, ssem2)

        def start_loads(i, s):
            base = base0 + i * CHUNK
            pltpu.async_copy(recv_hbm.at[pl.ds(base, CHUNK)], ibufs[s],
                             lsems[s])
            pltpu.async_copy(ea_hbm.at[pl.ds(base, CHUNK)], ebufs[s],
                             lsems[s])

        def wait_loads(i, s):
            base = base0 + i * CHUNK
            pltpu.make_async_copy(recv_hbm.at[pl.ds(base, CHUNK)], ibufs[s],
                                  lsems[s]).wait()
            pltpu.make_async_copy(ea_hbm.at[pl.ds(base, CHUNK)], ebufs[s],
                                  lsems[s]).wait()

        def start_scatter(s):
            pltpu.async_copy(ebufs[s], acc.at[ibufs[s]], csems[s], add=True)

        def wait_scatter(s):
            pltpu.make_async_copy(ebufs[s], acc.at[ibufs[s]],
                                  csems[s]).wait()

        def start_store(i, s):
            base = base0 + i * CHUNK
            pltpu.async_copy(ebufs[s], eout_hbm.at[pl.ds(base, CHUNK)],
                             ssems[s])

        def wait_store(i, s):
            base = base0 + i * CHUNK
            pltpu.make_async_copy(ebufs[s], eout_hbm.at[pl.ds(base, CHUNK)],
                                  ssems[s]).wait()

        for s in range(NSLOT):
            start_loads(s, s)

        for j in range(ZROWS):
            for k in range(D // 16):
                zbuf[j, pl.ds(k * 16, 16)] = jnp.zeros((16,), jnp.float32)

        nz = ROWS_PER_TILE // ZROWS

        def zfire(r, carry):
            pltpu.async_copy(zbuf, acc.at[pl.ds(row0 + r * ZROWS, ZROWS)],
                             csem0)
            return carry

        lax.fori_loop(0, nz, zfire, 0)

        def zdrain(r, carry):
            pltpu.make_async_copy(zbuf, acc.at[pl.ds(row0 + r * ZROWS, ZROWS)],
                                  csem0).wait()
            return carry

        lax.fori_loop(0, nz, zdrain, 0)
        plsc.subcore_barrier()

        def ring_body(t, carry):
            c = NSLOT * t
            for s in range(NSLOT):
                wait_loads(c + s, s)
                start_scatter(s)
                start_store(c + s, s)
            for s in range(NSLOT):
                wait_scatter(s)
                wait_store(c + s, s)

                @pl.when(c + s + NSLOT < N_CHUNKS)
                def _(s=s, c=c):
                    start_loads(c + s + NSLOT, s)

            return carry

        lax.fori_loop(0, N_CHUNKS // NSLOT, ring_body, 0)
        tbase = base0 + N_CHUNKS * CHUNK
        pltpu.sync_copy(recv_hbm.at[pl.ds(tbase, TAIL)], tibuf)
        pltpu.sync_copy(ea_hbm.at[pl.ds(tbase, TAIL)], tbuf)
        pltpu.sync_copy(tbuf, acc.at[tibuf], add=True)
        pltpu.sync_copy(tbuf, eout_hbm.at[pl.ds(tbase, TAIL)])
        plsc.subcore_barrier()

        pltpu.sync_copy(acc.at[pl.ds(row0, ROWS_PER_TILE)],
                        out_hbm.at[cid, pl.ds(row0, ROWS_PER_TILE)])

    return body(edge_attr, receivers)


BLK = 1000


def _mlp_body(x_ref, p_ref, w0_ref, b0_ref, w1_ref, b1_ref, w2_ref, b2_ref,
              o_ref):
    agg = p_ref[0] + p_ref[1]
    inp = jnp.concatenate([x_ref[...], agg], axis=-1)
    h = jnp.dot(inp, w0_ref[...], preferred_element_type=jnp.float32)
    h = jnp.maximum(h + b0_ref[...], 0.0)
    h = jnp.dot(h, w1_ref[...], preferred_element_type=jnp.float32)
    h = jnp.maximum(h + b1_ref[...], 0.0)
    h = jnp.dot(h, w2_ref[...], preferred_element_type=jnp.float32)
    h = h + b2_ref[...]
    mean = jnp.mean(h, axis=-1, keepdims=True)
    var = jnp.mean((h - mean) ** 2, axis=-1, keepdims=True)
    o_ref[...] = (h - mean) * lax.rsqrt(var + 1e-5)


def _tc_mlp(x, partials, W0, b0, W1, b1, W2, b2):
    grid = N_NODES // BLK
    return pl.pallas_call(
        _mlp_body,
        grid=(grid,),
        in_specs=[
            pl.BlockSpec((BLK, D), lambda i: (i, 0)),
            pl.BlockSpec((NC, BLK, D), lambda i: (0, i, 0)),
            pl.BlockSpec((2 * D, D), lambda i: (0, 0)),
            pl.BlockSpec((D,), lambda i: (0,)),
            pl.BlockSpec((D, D), lambda i: (0, 0)),
            pl.BlockSpec((D,), lambda i: (0,)),
            pl.BlockSpec((D, D), lambda i: (0, 0)),
            pl.BlockSpec((D,), lambda i: (0,)),
        ],
        out_specs=pl.BlockSpec((BLK, D), lambda i: (i, 0)),
        out_shape=jax.ShapeDtypeStruct((N_NODES, D), jnp.float32),
    )(x, partials, W0, b0, W1, b1, W2, b2)


def kernel(x, edge_attr, receivers, senders, W0, b0, W1, b1, W2, b2):
    partials, edge_attr_out = _sc_scatter_partials(
        edge_attr, receivers.astype(jnp.int32))
    updated_nodes = _tc_mlp(x, partials, W0, b0, W1, b1, W2, b2)
    return (updated_nodes, edge_attr_out, receivers, senders)


# speedup vs baseline: 6.6763x; 1.0121x over previous
import functools

import jax
import jax.numpy as jnp
from jax import lax
from jax.experimental import pallas as pl
from jax.experimental.pallas import tpu as pltpu
from jax.experimental.pallas import tpu_sc as plsc

N_NODES = 10000
N_EDGES = 320000
D = 128

NC = 2
NS = 16
NW = NC * NS
EDGES_PER_TILE = N_EDGES // NW
CHUNK = 104
N_CHUNKS = EDGES_PER_TILE // CHUNK
TAIL = EDGES_PER_TILE - N_CHUNKS * CHUNK
NSLOT = 3
N_PAD = 10240
ROWS_PER_TILE = N_PAD // NS
ZROWS = 32


def _sc_scatter_partials(edge_attr, receivers):
    mesh = plsc.VectorSubcoreMesh(core_axis_name="c", subcore_axis_name="s")

    @functools.partial(
        pl.kernel,
        out_type=(jax.ShapeDtypeStruct((NC, N_PAD, D), jnp.float32),
                  jax.ShapeDtypeStruct((N_EDGES, D), jnp.float32)),
        mesh=mesh,
        scratch_types=(
            [pltpu.VMEM((CHUNK, D), jnp.float32)] * NSLOT
            + [pltpu.VMEM((CHUNK,), jnp.int32)] * NSLOT
            + [pltpu.SemaphoreType.DMA] * NSLOT
            + [pltpu.SemaphoreType.DMA] * NSLOT
            + [pltpu.SemaphoreType.DMA] * NSLOT
            + [
                pltpu.VMEM((TAIL, D), jnp.float32),
                pltpu.VMEM((TAIL,), jnp.int32),
                pltpu.VMEM((ZROWS, D), jnp.float32),
                pltpu.VMEM_SHARED((N_PAD, D), jnp.float32),
            ]
        ),
    )
    def body(ea_hbm, recv_hbm, out_hbm, eout_hbm,
             eb0, eb1, eb2, ib0, ib1, ib2, lsem0, lsem1, lsem2,
             csem0, csem1, csem2, ssem0, ssem1, ssem2,
             tbuf, tibuf, zbuf, acc):
        cid = lax.axis_index("c")
        sid = lax.axis_index("s")
        wid = cid * NS + sid
        row0 = sid * ROWS_PER_TILE
        base0 = wid * EDGES_PER_TILE
        ebufs = (eb0, eb1, eb2)
        ibufs = (ib0, ib1, ib2)
        lsems = (lsem0, lsem1, lsem2)
        csems = (csem0, csem1, csem2)
        ssems = (ssem0, ssem1, ssem2)

        def start_loads(i, s):
            base = base0 + i * CHUNK
            pltpu.async_copy(recv_hbm.at[pl.ds(base, CHUNK)], ibufs[s],
                             lsems[s])
            pltpu.async_copy(ea_hbm.at[pl.ds(base, CHUNK)], ebufs[s],
                             lsems[s])

        def wait_loads(i, s):
            base = base0 + i * CHUNK
            pltpu.make_async_copy(recv_hbm.at[pl.ds(base, CHUNK)], ibufs[s],
                                  lsems[s]).wait()
            pltpu.make_async_copy(ea_hbm.at[pl.ds(base, CHUNK)], ebufs[s],
                                  lsems[s]).wait()

        def start_scatter(s):
            pltpu.async_copy(ebufs[s], acc.at[ibufs[s]], csems[s], add=True)

        def wait_scatter(s):
            pltpu.make_async_copy(ebufs[s], acc.at[ibufs[s]],
                                  csems[s]).wait()

        def start_store(i, s):
            base = base0 + i * CHUNK
            pltpu.async_copy(ebufs[s], eout_hbm.at[pl.ds(base, CHUNK)],
                             ssems[s])

        def wait_store(i, s):
            base = base0 + i * CHUNK
            pltpu.make_async_copy(ebufs[s], eout_hbm.at[pl.ds(base, CHUNK)],
                                  ssems[s]).wait()

        for s in range(NSLOT):
            start_loads(s, s)

        for j in range(ZROWS):
            for k in range(D // 16):
                zbuf[j, pl.ds(k * 16, 16)] = jnp.zeros((16,), jnp.float32)

        nz = ROWS_PER_TILE // ZROWS

        def zfire(r, carry):
            pltpu.async_copy(zbuf, acc.at[pl.ds(row0 + r * ZROWS, ZROWS)],
                             csem0)
            return carry

        lax.fori_loop(0, nz, zfire, 0)

        def zdrain(r, carry):
            pltpu.make_async_copy(zbuf, acc.at[pl.ds(row0 + r * ZROWS, ZROWS)],
                                  csem0).wait()
            return carry

        lax.fori_loop(0, nz, zdrain, 0)
        plsc.subcore_barrier()

        def ring_body(t, carry):
            c = NSLOT * t
            for s in range(NSLOT):
                wait_loads(c + s, s)
                start_scatter(s)
                start_store(c + s, s)
            for s in range(NSLOT):
                wait_scatter(s)
                wait_store(c + s, s)

                @pl.when(c + s + NSLOT < N_CHUNKS)
                def _(s=s, c=c):
                    start_loads(c + s + NSLOT, s)

            return carry

        lax.fori_loop(0, N_CHUNKS // NSLOT, ring_body, 0)
        tbase = base0 + N_CHUNKS * CHUNK
        pltpu.sync_copy(recv_hbm.at[pl.ds(tbase, TAIL)], tibuf)
        pltpu.sync_copy(ea_hbm.at[pl.ds(tbase, TAIL)], tbuf)
        pltpu.sync_copy(tbuf, acc.at[tibuf], add=True)
        pltpu.sync_copy(tbuf, eout_hbm.at[pl.ds(tbase, TAIL)])
        plsc.subcore_barrier()

        pltpu.sync_copy(acc.at[pl.ds(row0, ROWS_PER_TILE)],
                        out_hbm.at[cid, pl.ds(row0, ROWS_PER_TILE)])

    return body(edge_attr, receivers)


BLK = 2000


def _mlp_body(x_ref, p_ref, w0_ref, b0_ref, w1_ref, b1_ref, w2_ref, b2_ref,
              o_ref):
    agg = p_ref[0] + p_ref[1]
    inp = jnp.concatenate([x_ref[...], agg], axis=-1)
    h = jnp.dot(inp, w0_ref[...], preferred_element_type=jnp.float32)
    h = jnp.maximum(h + b0_ref[...], 0.0)
    h = jnp.dot(h, w1_ref[...], preferred_element_type=jnp.float32)
    h = jnp.maximum(h + b1_ref[...], 0.0)
    h = jnp.dot(h, w2_ref[...], preferred_element_type=jnp.float32)
    h = h + b2_ref[...]
    mean = jnp.mean(h, axis=-1, keepdims=True)
    var = jnp.mean((h - mean) ** 2, axis=-1, keepdims=True)
    o_ref[...] = (h - mean) * lax.rsqrt(var + 1e-5)


def _tc_mlp(x, partials, W0, b0, W1, b1, W2, b2):
    grid = N_NODES // BLK
    return pl.pallas_call(
        _mlp_body,
        grid=(grid,),
        in_specs=[
            pl.BlockSpec((BLK, D), lambda i: (i, 0)),
            pl.BlockSpec((NC, BLK, D), lambda i: (0, i, 0)),
            pl.BlockSpec((2 * D, D), lambda i: (0, 0)),
            pl.BlockSpec((D,), lambda i: (0,)),
            pl.BlockSpec((D, D), lambda i: (0, 0)),
            pl.BlockSpec((D,), lambda i: (0,)),
            pl.BlockSpec((D, D), lambda i: (0, 0)),
            pl.BlockSpec((D,), lambda i: (0,)),
        ],
        out_specs=pl.BlockSpec((BLK, D), lambda i: (i, 0)),
        out_shape=jax.ShapeDtypeStruct((N_NODES, D), jnp.float32),
    )(x, partials, W0, b0, W1, b1, W2, b2)


def kernel(x, edge_attr, receivers, senders, W0, b0, W1, b1, W2, b2):
    partials, edge_attr_out = _sc_scatter_partials(
        edge_attr, receivers.astype(jnp.int32))
    updated_nodes = _tc_mlp(x, partials, W0, b0, W1, b1, W2, b2)
    return (updated_nodes, edge_attr_out, receivers, senders)


# speedup vs baseline: 6.9609x; 1.0426x over previous
import functools

import jax
import jax.numpy as jnp
from jax import lax
from jax.experimental import pallas as pl
from jax.experimental.pallas import tpu as pltpu
from jax.experimental.pallas import tpu_sc as plsc

N_NODES = 10000
N_EDGES = 320000
D = 128

NC = 2
NS = 16
NW = NC * NS
EDGES_PER_TILE = N_EDGES // NW
CHUNK = 64
N_CHUNKS = EDGES_PER_TILE // CHUNK
TAIL = EDGES_PER_TILE - N_CHUNKS * CHUNK
NSLOT = 4
N_PAD = 10240
ROWS_PER_TILE = N_PAD // NS
ZROWS = 32


def _sc_scatter_partials(edge_attr, receivers):
    mesh = plsc.VectorSubcoreMesh(core_axis_name="c", subcore_axis_name="s")

    @functools.partial(
        pl.kernel,
        out_type=(jax.ShapeDtypeStruct((NC, N_PAD, D), jnp.float32),
                  jax.ShapeDtypeStruct((N_EDGES, D), jnp.float32)),
        mesh=mesh,
        scratch_types=(
            [pltpu.VMEM((CHUNK, D), jnp.float32)] * NSLOT
            + [pltpu.VMEM((CHUNK,), jnp.int32)] * NSLOT
            + [pltpu.SemaphoreType.DMA] * NSLOT
            + [pltpu.SemaphoreType.DMA] * NSLOT
            + [pltpu.SemaphoreType.DMA] * NSLOT
            + [
                pltpu.VMEM((TAIL, D), jnp.float32),
                pltpu.VMEM((TAIL,), jnp.int32),
                pltpu.VMEM((ZROWS, D), jnp.float32),
                pltpu.VMEM_SHARED((N_PAD, D), jnp.float32),
            ]
        ),
    )
    def body(ea_hbm, recv_hbm, out_hbm, eout_hbm,
             eb0, eb1, eb2, eb3, ib0, ib1, ib2, ib3, lsem0, lsem1, lsem2,
             lsem3, csem0, csem1, csem2, csem3, ssem0, ssem1, ssem2, ssem3,
             tbuf, tibuf, zbuf, acc):
        cid = lax.axis_index("c")
        sid = lax.axis_index("s")
        wid = cid * NS + sid
        row0 = sid * ROWS_PER_TILE
        base0 = wid * EDGES_PER_TILE
        ebufs = (eb0, eb1, eb2, eb3)
        ibufs = (ib0, ib1, ib2, ib3)
        lsems = (lsem0, lsem1, lsem2, lsem3)
        csems = (csem0, csem1, csem2, csem3)
        ssems = (ssem0, ssem1, ssem2, ssem3)

        def start_loads(i, s):
            base = base0 + i * CHUNK
            pltpu.async_copy(recv_hbm.at[pl.ds(base, CHUNK)], ibufs[s],
                             lsems[s])
            pltpu.async_copy(ea_hbm.at[pl.ds(base, CHUNK)], ebufs[s],
                             lsems[s])

        def wait_loads(i, s):
            base = base0 + i * CHUNK
            pltpu.make_async_copy(recv_hbm.at[pl.ds(base, CHUNK)], ibufs[s],
                                  lsems[s]).wait()
            pltpu.make_async_copy(ea_hbm.at[pl.ds(base, CHUNK)], ebufs[s],
                                  lsems[s]).wait()

        def start_scatter(s):
            pltpu.async_copy(ebufs[s], acc.at[ibufs[s]], csems[s], add=True)

        def wait_scatter(s):
            pltpu.make_async_copy(ebufs[s], acc.at[ibufs[s]],
                                  csems[s]).wait()

        def start_store(i, s):
            base = base0 + i * CHUNK
            pltpu.async_copy(ebufs[s], eout_hbm.at[pl.ds(base, CHUNK)],
                             ssems[s])

        def wait_store(i, s):
            base = base0 + i * CHUNK
            pltpu.make_async_copy(ebufs[s], eout_hbm.at[pl.ds(base, CHUNK)],
                                  ssems[s]).wait()

        for s in range(NSLOT):
            start_loads(s, s)

        for j in range(ZROWS):
            for k in range(D // 16):
                zbuf[j, pl.ds(k * 16, 16)] = jnp.zeros((16,), jnp.float32)

        nz = ROWS_PER_TILE // ZROWS

        def zfire(r, carry):
            pltpu.async_copy(zbuf, acc.at[pl.ds(row0 + r * ZROWS, ZROWS)],
                             csem0)
            return carry

        lax.fori_loop(0, nz, zfire, 0)

        def zdrain(r, carry):
            pltpu.make_async_copy(zbuf, acc.at[pl.ds(row0 + r * ZROWS, ZROWS)],
                                  csem0).wait()
            return carry

        lax.fori_loop(0, nz, zdrain, 0)
        plsc.subcore_barrier()

        def ring_body(t, carry):
            c = NSLOT * t
            for s in range(NSLOT):
                wait_loads(c + s, s)
                start_scatter(s)
                start_store(c + s, s)
            for s in range(NSLOT):
                wait_scatter(s)
                wait_store(c + s, s)

                @pl.when(c + s + NSLOT < N_CHUNKS)
                def _(s=s, c=c):
                    start_loads(c + s + NSLOT, s)

            return carry

        lax.fori_loop(0, N_CHUNKS // NSLOT, ring_body, 0)
        tbase = base0 + N_CHUNKS * CHUNK
        pltpu.sync_copy(recv_hbm.at[pl.ds(tbase, TAIL)], tibuf)
        pltpu.sync_copy(ea_hbm.at[pl.ds(tbase, TAIL)], tbuf)
        pltpu.sync_copy(tbuf, acc.at[tibuf], add=True)
        pltpu.sync_copy(tbuf, eout_hbm.at[pl.ds(tbase, TAIL)])
        plsc.subcore_barrier()

        pltpu.sync_copy(acc.at[pl.ds(row0, ROWS_PER_TILE)],
                        out_hbm.at[cid, pl.ds(row0, ROWS_PER_TILE)])

    return body(edge_attr, receivers)


BLK = 2000


def _mlp_body(x_ref, p_ref, w0_ref, b0_ref, w1_ref, b1_ref, w2_ref, b2_ref,
              o_ref):
    agg = p_ref[0] + p_ref[1]
    inp = jnp.concatenate([x_ref[...], agg], axis=-1)
    h = jnp.dot(inp, w0_ref[...], preferred_element_type=jnp.float32)
    h = jnp.maximum(h + b0_ref[...], 0.0)
    h = jnp.dot(h, w1_ref[...], preferred_element_type=jnp.float32)
    h = jnp.maximum(h + b1_ref[...], 0.0)
    h = jnp.dot(h, w2_ref[...], preferred_element_type=jnp.float32)
    h = h + b2_ref[...]
    mean = jnp.mean(h, axis=-1, keepdims=True)
    var = jnp.mean((h - mean) ** 2, axis=-1, keepdims=True)
    o_ref[...] = (h - mean) * lax.rsqrt(var + 1e-5)


def _tc_mlp(x, partials, W0, b0, W1, b1, W2, b2):
    grid = N_NODES // BLK
    return pl.pallas_call(
        _mlp_body,
        grid=(grid,),
        in_specs=[
            pl.BlockSpec((BLK, D), lambda i: (i, 0)),
            pl.BlockSpec((NC, BLK, D), lambda i: (0, i, 0)),
            pl.BlockSpec((2 * D, D), lambda i: (0, 0)),
            pl.BlockSpec((D,), lambda i: (0,)),
            pl.BlockSpec((D, D), lambda i: (0, 0)),
            pl.BlockSpec((D,), lambda i: (0,)),
            pl.BlockSpec((D, D), lambda i: (0, 0)),
            pl.BlockSpec((D,), lambda i: (0,)),
        ],
        out_specs=pl.BlockSpec((BLK, D), lambda i: (i, 0)),
        out_shape=jax.ShapeDtypeStruct((N_NODES, D), jnp.float32),
    )(x, partials, W0, b0, W1, b1, W2, b2)


def kernel(x, edge_attr, receivers, senders, W0, b0, W1, b1, W2, b2):
    partials, edge_attr_out = _sc_scatter_partials(
        edge_attr, receivers.astype(jnp.int32))
    updated_nodes = _tc_mlp(x, partials, W0, b0, W1, b1, W2, b2)
    return (updated_nodes, edge_attr_out, receivers, senders)
